# baseline (speedup 1.0000x reference)
import jax
import jax.numpy as jnp
from jax.experimental import pallas as pl

_GRID = 10


def _copy_x_body(xs_ref, xt_ref, u_ref, oxs_ref, oxt_ref, ou_ref):
    oxs_ref[...] = xs_ref[...]
    oxt_ref[...] = xt_ref[...]

    @pl.when(pl.program_id(0) == 0)
    def _():
        ou_ref[...] = u_ref[...]


def _copy_e_body(ea_ref, oea_ref):
    oea_ref[...] = ea_ref[...]


def kernel(x_s, x_t, edge_index, edge_attr, u, batch_e, batch_s, batch_t):
    del edge_index, batch_e, batch_s, batch_t
    n_s, d_feat = x_s.shape
    e, d_edge = edge_attr.shape
    bx = n_s // _GRID
    be = e // _GRID

    xspecs = [
        pl.BlockSpec((bx, d_feat), lambda i: (i, 0)),
        pl.BlockSpec((bx, d_feat), lambda i: (i, 0)),
        pl.BlockSpec(u.shape, lambda i: (0, 0)),
    ]
    xs_o, xt_o, u_o = pl.pallas_call(
        _copy_x_body,
        grid=(_GRID,),
        in_specs=xspecs,
        out_specs=xspecs,
        out_shape=[
            jax.ShapeDtypeStruct(x_s.shape, x_s.dtype),
            jax.ShapeDtypeStruct(x_t.shape, x_t.dtype),
            jax.ShapeDtypeStruct(u.shape, u.dtype),
        ],
    )(x_s, x_t, u)

    espec = [pl.BlockSpec((be, d_edge), lambda i: (i, 0))]
    ea_o, = pl.pallas_call(
        _copy_e_body,
        grid=(_GRID,),
        in_specs=espec,
        out_specs=espec,
        out_shape=[jax.ShapeDtypeStruct(edge_attr.shape, edge_attr.dtype)],
    )(edge_attr)

    return (xs_o, xt_o, ea_o, u_o)
